# baseline (speedup 1.0000x reference)
import functools

import jax
import jax.numpy as jnp
from jax.experimental import pallas as pl
from jax.experimental.pallas import tpu as pltpu

_BR = 16


def _hinge_body(y_ref, x_ref, o_ref):
    x = x_ref[...]
    yv = y_ref[...]
    c = x.shape[1]
    cols = jax.lax.broadcasted_iota(jnp.int32, x.shape, 1)
    oy = jnp.sum(jnp.where(cols == yv, x, 0.0), axis=1, keepdims=True)
    s = jnp.sum(jnp.maximum(x - (oy - 1.0), 0.0), axis=1, keepdims=True)
    o_ref[...] = (s - 1.0) / c


def kernel(output, y):
    b, c = output.shape
    y2 = y.astype(jnp.int32).reshape(b, 1)
    out = pl.pallas_call(
        _hinge_body,
        grid=(b // _BR,),
        in_specs=[
            pl.BlockSpec((_BR, 1), lambda i: (i, 0)),
            pl.BlockSpec((_BR, c), lambda i: (i, 0)),
        ],
        out_specs=pl.BlockSpec((_BR, 1), lambda i: (i, 0)),
        out_shape=jax.ShapeDtypeStruct((b, 1), jnp.float32),
    )(y2, output)
    return out.reshape(b)


# speedup vs baseline: 1.0803x; 1.0803x over previous
import functools

import jax
import jax.numpy as jnp
from jax.experimental import pallas as pl
from jax.experimental.pallas import tpu as pltpu

_BR = 64


def _hinge_body(y_ref, x_ref, o_ref):
    x = x_ref[...]
    yv = y_ref[...]
    c = x.shape[1]
    cols = jax.lax.broadcasted_iota(jnp.int32, x.shape, 1)
    oy = jnp.sum(jnp.where(cols == yv, x, 0.0), axis=1, keepdims=True)
    s = jnp.sum(jnp.maximum(x - (oy - 1.0), 0.0), axis=1, keepdims=True)
    o_ref[...] = (s - 1.0) / c


def kernel(output, y):
    b, c = output.shape
    y2 = y.astype(jnp.int32).reshape(b, 1)
    out = pl.pallas_call(
        _hinge_body,
        grid=(b // _BR,),
        in_specs=[
            pl.BlockSpec((_BR, 1), lambda i: (i, 0)),
            pl.BlockSpec((_BR, c), lambda i: (i, 0)),
        ],
        out_specs=pl.BlockSpec((_BR, 1), lambda i: (i, 0)),
        out_shape=jax.ShapeDtypeStruct((b, 1), jnp.float32),
    )(y2, output)
    return out.reshape(b)
